# baseline (speedup 1.0000x reference)
import functools

import jax
import jax.numpy as jnp
from jax import lax
from jax.experimental import pallas as pl
from jax.experimental.pallas import tpu as pltpu
from jax.experimental.pallas import tpu_sc as plsc

_NC = 2
_NS = 16
_NW = _NC * _NS
_BLK = 512
_D = 16


def _make_sc_scatter(n_nodes_p, k_blocks, compute_deg):
    assert k_blocks % 3 == 0
    rpt = n_nodes_p // _NS
    mesh = plsc.VectorSubcoreMesh(
        core_axis_name="c", subcore_axis_name="s",
        num_cores=_NC, num_subcores=_NS)

    acc_t = jax.ShapeDtypeStruct((_NC, n_nodes_p, _D), jnp.float32)
    out_type = [acc_t, acc_t] if compute_deg else acc_t

    scratch_types = [
        pltpu.VMEM((k_blocks + 2, _BLK), jnp.int32),
        pltpu.VMEM((k_blocks, _BLK), jnp.int32),
        pltpu.VMEM((3, _BLK, _D), jnp.float32),
        pltpu.VMEM((_BLK, _D), jnp.float32),
        pltpu.VMEM_SHARED((n_nodes_p, _D), jnp.float32),
        pltpu.VMEM_SHARED((n_nodes_p, _D), jnp.float32),
        pltpu.SemaphoreType.DMA,
        pltpu.SemaphoreType.DMA,
        pltpu.SemaphoreType.DMA,
    ]

    def body(tbl, srcb, dstb, zeros, ones, *refs):
        if compute_deg:
            out_acc, out_deg = refs[0], refs[1]
            rest = refs[2:]
        else:
            out_acc, out_deg = refs[0], None
            rest = refs[1:]
        (src_v, dst_v, rows_v, ones_v, acc_sh, deg_sh,
         sem_g0, sem_g1, sem_g2) = rest
        sem_g = (sem_g0, sem_g1, sem_g2)

        cid = lax.axis_index("c")
        sid = lax.axis_index("s")
        wid = cid * _NS + sid

        pltpu.sync_copy(srcb.at[wid], src_v.at[pl.ds(0, k_blocks)])
        pltpu.sync_copy(dstb.at[wid], dst_v)
        pltpu.sync_copy(ones, ones_v)
        zi = jnp.zeros((16,), jnp.int32)
        for r in range(k_blocks, k_blocks + 2):
            for q in range(_BLK // 16):
                src_v[r, pl.ds(q * 16, 16)] = zi

        r0 = sid * rpt
        pltpu.sync_copy(zeros.at[pl.ds(r0, rpt)], acc_sh.at[pl.ds(r0, rpt)])
        if compute_deg:
            pltpu.sync_copy(zeros.at[pl.ds(r0, rpt)], deg_sh.at[pl.ds(r0, rpt)])
        plsc.subcore_barrier()

        def fire_gather(j, buf, sem):
            pltpu.async_copy(tbl.at[src_v.at[j]], rows_v.at[buf], sem)

        def drain_gather(buf, sem):
            pltpu.make_async_copy(
                tbl.at[pl.ds(0, _BLK)], rows_v.at[buf], sem).wait()

        fire_gather(0, 0, sem_g[0])
        fire_gather(1, 1, sem_g[1])

        def group(i):
            base = i * 3
            for b in range(3):
                j = base + b
                nb = (b + 2) % 3
                fire_gather(j + 2, nb, sem_g[nb])
                drain_gather(b, sem_g[b])
                pltpu.sync_copy(rows_v.at[b], acc_sh.at[dst_v.at[j]],
                                add=True)
                if compute_deg:
                    pltpu.sync_copy(ones_v, deg_sh.at[dst_v.at[j]],
                                    add=True)

        lax.fori_loop(0, k_blocks // 3, lambda i, c: (group(i), c)[1], 0)

        drain_gather(k_blocks % 3, sem_g[k_blocks % 3])
        drain_gather((k_blocks + 1) % 3, sem_g[(k_blocks + 1) % 3])
        plsc.subcore_barrier()

        pltpu.sync_copy(acc_sh.at[pl.ds(r0, rpt)],
                        out_acc.at[cid, pl.ds(r0, rpt)])
        if compute_deg:
            pltpu.sync_copy(deg_sh.at[pl.ds(r0, rpt)],
                            out_deg.at[cid, pl.ds(r0, rpt)])

    return pl.kernel(body, out_type=out_type, mesh=mesh,
                     scratch_types=scratch_types,
                     compiler_params=pltpu.CompilerParams(
                         use_tc_tiling_on_sc=False))


def _proj_body(x_ref, wl_ref, wr_ref, p_ref, q_ref):
    xb = x_ref[...]
    p_ref[...] = jnp.dot(xb, wl_ref[...], preferred_element_type=jnp.float32)
    q_ref[...] = jnp.dot(xb, wr_ref[...], preferred_element_type=jnp.float32)


def _h_body(acc_ref, dega_ref, q_ref, b_ref, h_ref, deg_ref):
    a = acc_ref[0] + acc_ref[1]
    dg = jnp.maximum(dega_ref[0] + dega_ref[1], 1.0)
    h_ref[...] = jnp.maximum(a / dg + q_ref[...] + b_ref[...], 0.0)
    deg_ref[...] = dg


def _out_body(acc_ref, deg_ref, h_ref, wl_ref, wr_ref, b_ref, o_ref):
    mean2 = (acc_ref[0] + acc_ref[1]) / deg_ref[...]
    z = (jnp.dot(mean2, wl_ref[...], preferred_element_type=jnp.float32)
         + jnp.dot(h_ref[...], wr_ref[...], preferred_element_type=jnp.float32)
         + b_ref[...])
    m = jnp.max(z, axis=1, keepdims=True)
    s = jnp.sum(jnp.exp(z - m), axis=1, keepdims=True)
    o_ref[...] = z - m - jnp.log(s)


def kernel(x, edge_index, W1_l, W1_r, b1, W2_l, W2_r, b2):
    n, d_feat = x.shape
    d_hid = W1_l.shape[1]
    n_cls = W2_l.shape[1]
    e = edge_index.shape[1]
    assert d_hid == _D

    src = edge_index[0].astype(jnp.int32)
    dst = edge_index[1].astype(jnp.int32)

    k_blocks = -(-(-(-e // (_NW * _BLK))) // 3) * 3
    e_pad = _NW * k_blocks * _BLK - e
    src_p = jnp.concatenate(
        [src, jnp.zeros((e_pad,), jnp.int32)]).reshape(_NW, k_blocks, _BLK)
    dst_p = jnp.concatenate(
        [dst, jnp.full((e_pad,), n, jnp.int32)]).reshape(_NW, k_blocks, _BLK)

    n_nodes_p = -(-(n + 1) // (_NS * 8)) * (_NS * 8)
    zeros = jnp.zeros((n_nodes_p, _D), jnp.float32)
    ones = jnp.ones((_BLK, _D), jnp.float32)

    sc_scatter_deg = _make_sc_scatter(n_nodes_p, k_blocks, True)
    sc_scatter = _make_sc_scatter(n_nodes_p, k_blocks, False)

    rb = 1000
    grid = (n // rb,)

    p1, q1 = pl.pallas_call(
        _proj_body,
        grid=grid,
        in_specs=[
            pl.BlockSpec((rb, d_feat), lambda i: (i, 0)),
            pl.BlockSpec((d_feat, d_hid), lambda i: (0, 0)),
            pl.BlockSpec((d_feat, d_hid), lambda i: (0, 0)),
        ],
        out_specs=[pl.BlockSpec((rb, d_hid), lambda i: (i, 0))] * 2,
        out_shape=[jax.ShapeDtypeStruct((n, d_hid), jnp.float32)] * 2,
    )(x, W1_l, W1_r)

    acc1, dega = sc_scatter_deg(p1, src_p, dst_p, zeros, ones)
    acc1 = acc1[:, :n, :]
    dega = dega[:, :n, :]

    h, deg = pl.pallas_call(
        _h_body,
        grid=grid,
        in_specs=[
            pl.BlockSpec((_NC, rb, d_hid), lambda i: (0, i, 0)),
            pl.BlockSpec((_NC, rb, d_hid), lambda i: (0, i, 0)),
            pl.BlockSpec((rb, d_hid), lambda i: (i, 0)),
            pl.BlockSpec((1, d_hid), lambda i: (0, 0)),
        ],
        out_specs=[pl.BlockSpec((rb, d_hid), lambda i: (i, 0))] * 2,
        out_shape=[jax.ShapeDtypeStruct((n, d_hid), jnp.float32)] * 2,
    )(acc1, dega, q1, b1.reshape(1, d_hid))

    acc2 = sc_scatter(h, src_p, dst_p, zeros, ones)
    acc2 = acc2[:, :n, :]

    out = pl.pallas_call(
        _out_body,
        grid=grid,
        in_specs=[
            pl.BlockSpec((_NC, rb, d_hid), lambda i: (0, i, 0)),
            pl.BlockSpec((rb, d_hid), lambda i: (i, 0)),
            pl.BlockSpec((rb, d_hid), lambda i: (i, 0)),
            pl.BlockSpec((d_hid, n_cls), lambda i: (0, 0)),
            pl.BlockSpec((d_hid, n_cls), lambda i: (0, 0)),
            pl.BlockSpec((1, n_cls), lambda i: (0, 0)),
        ],
        out_specs=pl.BlockSpec((rb, n_cls), lambda i: (i, 0)),
        out_shape=jax.ShapeDtypeStruct((n, n_cls), jnp.float32),
    )(acc2, deg, h, W2_l, W2_r, b2.reshape(1, n_cls))

    return out


# speedup vs baseline: 2.3396x; 2.3396x over previous
import functools

import jax
import jax.numpy as jnp
from jax import lax
from jax.experimental import pallas as pl
from jax.experimental.pallas import tpu as pltpu
from jax.experimental.pallas import tpu_sc as plsc

_NC = 2
_NS = 16
_NW = _NC * _NS
_BLK = 128
_D = 16


def _make_sc_scatter(n_nodes_p, k_blocks, compute_deg):
    rpt = n_nodes_p // _NS
    hrows = n_nodes_p // 16
    mesh = plsc.VectorSubcoreMesh(
        core_axis_name="c", subcore_axis_name="s",
        num_cores=_NC, num_subcores=_NS)

    acc_t = jax.ShapeDtypeStruct((_NC, n_nodes_p, _D), jnp.float32)
    deg_t = jax.ShapeDtypeStruct((_NW, hrows, 16), jnp.float32)
    out_type = [acc_t, deg_t] if compute_deg else acc_t

    scratch_types = [
        pltpu.VMEM((k_blocks, _BLK), jnp.int32),
        pltpu.VMEM((k_blocks, _BLK), jnp.int32),
        pltpu.VMEM((_BLK, _D), jnp.float32),
        pltpu.VMEM((hrows, 16), jnp.float32),
        pltpu.VMEM_SHARED((n_nodes_p, _D), jnp.float32),
        pltpu.SemaphoreType.DMA,
    ]

    def body(tbl, srcb, dstb, zeros, *refs):
        if compute_deg:
            out_acc, out_deg = refs[0], refs[1]
            rest = refs[2:]
        else:
            out_acc, out_deg = refs[0], None
            rest = refs[1:]
        src_v, dst_v, rows_v, hist_v, acc_sh, sem = rest

        cid = lax.axis_index("c")
        sid = lax.axis_index("s")
        wid = cid * _NS + sid

        pltpu.sync_copy(srcb.at[wid], src_v)
        pltpu.sync_copy(dstb.at[wid], dst_v)
        if compute_deg:
            pltpu.sync_copy(zeros.at[pl.ds(0, hrows)], hist_v)

        r0 = sid * rpt
        pltpu.sync_copy(zeros.at[pl.ds(r0, rpt)], acc_sh.at[pl.ds(r0, rpt)])
        plsc.subcore_barrier()

        one_v = jnp.ones((16,), jnp.float32)

        def step(j, carry):
            pltpu.async_copy(tbl.at[src_v.at[j]], rows_v, sem).wait()
            pltpu.sync_copy(rows_v, acc_sh.at[dst_v.at[j]], add=True)
            if compute_deg:
                for q in range(_BLK // 16):
                    d = dst_v[j, pl.ds(q * 16, 16)]
                    plsc.addupdate_scatter(
                        hist_v,
                        [lax.shift_right_logical(d, 4),
                         lax.bitwise_and(d, 15)],
                        one_v)
            return carry

        lax.fori_loop(0, k_blocks, step, 0)
        plsc.subcore_barrier()

        pltpu.sync_copy(acc_sh.at[pl.ds(r0, rpt)],
                        out_acc.at[cid, pl.ds(r0, rpt)])
        if compute_deg:
            pltpu.sync_copy(hist_v, out_deg.at[wid])

    return pl.kernel(body, out_type=out_type, mesh=mesh,
                     scratch_types=scratch_types,
                     compiler_params=pltpu.CompilerParams(
                         use_tc_tiling_on_sc=False,
                         needs_layout_passes=False))


def _proj_body(x_ref, wl_ref, wr_ref, p_ref, q_ref):
    xb = x_ref[...]
    p_ref[...] = jnp.dot(xb, wl_ref[...], preferred_element_type=jnp.float32)
    q_ref[...] = jnp.dot(xb, wr_ref[...], preferred_element_type=jnp.float32)


def _h_body(acc_ref, dega_ref, q_ref, b_ref, h_ref, deg_ref):
    a = acc_ref[0] + acc_ref[1]
    dg = jnp.maximum(jnp.sum(dega_ref[...], axis=1), 1.0)[:, None]
    h_ref[...] = jnp.maximum(a / dg + q_ref[...] + b_ref[...], 0.0)
    deg_ref[...] = jnp.broadcast_to(dg, h_ref.shape)


def _out_body(acc_ref, deg_ref, h_ref, wl_ref, wr_ref, b_ref, o_ref):
    mean2 = (acc_ref[0] + acc_ref[1]) / deg_ref[...]
    z = (jnp.dot(mean2, wl_ref[...], preferred_element_type=jnp.float32)
         + jnp.dot(h_ref[...], wr_ref[...], preferred_element_type=jnp.float32)
         + b_ref[...])
    m = jnp.max(z, axis=1, keepdims=True)
    s = jnp.sum(jnp.exp(z - m), axis=1, keepdims=True)
    o_ref[...] = z - m - jnp.log(s)


def kernel(x, edge_index, W1_l, W1_r, b1, W2_l, W2_r, b2):
    n, d_feat = x.shape
    d_hid = W1_l.shape[1]
    n_cls = W2_l.shape[1]
    e = edge_index.shape[1]
    assert d_hid == _D

    src = edge_index[0].astype(jnp.int32)
    dst = edge_index[1].astype(jnp.int32)

    k_blocks = -(-e // (_NW * _BLK))
    e_pad = _NW * k_blocks * _BLK - e
    src_p = jnp.concatenate(
        [src, jnp.zeros((e_pad,), jnp.int32)]).reshape(_NW, k_blocks, _BLK)
    dst_p = jnp.concatenate(
        [dst, jnp.full((e_pad,), n, jnp.int32)]).reshape(_NW, k_blocks, _BLK)

    n_nodes_p = -(-(n + 1) // (_NS * 8)) * (_NS * 8)
    zeros = jnp.zeros((n_nodes_p, _D), jnp.float32)

    sc_scatter_deg = _make_sc_scatter(n_nodes_p, k_blocks, True)
    sc_scatter = _make_sc_scatter(n_nodes_p, k_blocks, False)

    rb = 1000
    grid = (n // rb,)

    p1, q1 = pl.pallas_call(
        _proj_body,
        grid=grid,
        in_specs=[
            pl.BlockSpec((rb, d_feat), lambda i: (i, 0)),
            pl.BlockSpec((d_feat, d_hid), lambda i: (0, 0)),
            pl.BlockSpec((d_feat, d_hid), lambda i: (0, 0)),
        ],
        out_specs=[pl.BlockSpec((rb, d_hid), lambda i: (i, 0))] * 2,
        out_shape=[jax.ShapeDtypeStruct((n, d_hid), jnp.float32)] * 2,
    )(x, W1_l, W1_r)

    acc1, degh = sc_scatter_deg(p1, src_p, dst_p, zeros)
    acc1 = acc1[:, :n, :]
    dega = degh.reshape(_NW, n_nodes_p)[:, :n].T

    h, deg = pl.pallas_call(
        _h_body,
        grid=grid,
        in_specs=[
            pl.BlockSpec((_NC, rb, d_hid), lambda i: (0, i, 0)),
            pl.BlockSpec((rb, _NW), lambda i: (i, 0)),
            pl.BlockSpec((rb, d_hid), lambda i: (i, 0)),
            pl.BlockSpec((1, d_hid), lambda i: (0, 0)),
        ],
        out_specs=[pl.BlockSpec((rb, d_hid), lambda i: (i, 0))] * 2,
        out_shape=[jax.ShapeDtypeStruct((n, d_hid), jnp.float32)] * 2,
    )(acc1, dega, q1, b1.reshape(1, d_hid))

    acc2 = sc_scatter(h, src_p, dst_p, zeros)
    acc2 = acc2[:, :n, :]

    out = pl.pallas_call(
        _out_body,
        grid=grid,
        in_specs=[
            pl.BlockSpec((_NC, rb, d_hid), lambda i: (0, i, 0)),
            pl.BlockSpec((rb, d_hid), lambda i: (i, 0)),
            pl.BlockSpec((rb, d_hid), lambda i: (i, 0)),
            pl.BlockSpec((d_hid, n_cls), lambda i: (0, 0)),
            pl.BlockSpec((d_hid, n_cls), lambda i: (0, 0)),
            pl.BlockSpec((1, n_cls), lambda i: (0, 0)),
        ],
        out_specs=pl.BlockSpec((rb, n_cls), lambda i: (i, 0)),
        out_shape=jax.ShapeDtypeStruct((n, n_cls), jnp.float32),
    )(acc2, deg, h, W2_l, W2_r, b2.reshape(1, n_cls))

    return out


# speedup vs baseline: 3.4828x; 1.4887x over previous
import functools

import jax
import jax.numpy as jnp
from jax import lax
from jax.experimental import pallas as pl
from jax.experimental.pallas import tpu as pltpu
from jax.experimental.pallas import tpu_sc as plsc

_NC = 2
_NS = 16
_NW = _NC * _NS
_BLK = 128
_D = 16


def _make_sc_scatter(n_nodes_p, k_blocks, compute_deg):
    rpt = n_nodes_p // _NS
    hrows = n_nodes_p // 16
    mesh = plsc.VectorSubcoreMesh(
        core_axis_name="c", subcore_axis_name="s",
        num_cores=_NC, num_subcores=_NS)

    acc_t = jax.ShapeDtypeStruct((_NC, n_nodes_p, _D), jnp.float32)
    deg_t = jax.ShapeDtypeStruct((_NW, hrows, 16), jnp.float32)
    out_type = [acc_t, deg_t] if compute_deg else acc_t

    scratch_types = [
        pltpu.VMEM((k_blocks, _BLK), jnp.int32),
        pltpu.VMEM((k_blocks, _BLK), jnp.int32),
        pltpu.VMEM((_BLK, _D), jnp.float32),
        pltpu.VMEM((hrows, 16), jnp.float32),
        pltpu.VMEM_SHARED((n_nodes_p, _D), jnp.float32),
        pltpu.VMEM_SHARED((n_nodes_p, _D), jnp.float32),
        pltpu.SemaphoreType.DMA,
    ]

    def body(tbl, srcb, dstb, zeros, *refs):
        if compute_deg:
            out_acc, out_deg = refs[0], refs[1]
            rest = refs[2:]
        else:
            out_acc, out_deg = refs[0], None
            rest = refs[1:]
        src_v, dst_v, rows_v, hist_v, acc_sh, tbl_sh, sem = rest

        cid = lax.axis_index("c")
        sid = lax.axis_index("s")
        wid = cid * _NS + sid

        pltpu.sync_copy(srcb.at[wid], src_v)
        pltpu.sync_copy(dstb.at[wid], dst_v)
        if compute_deg:
            pltpu.sync_copy(zeros.at[pl.ds(0, hrows)], hist_v)

        r0 = sid * rpt
        pltpu.sync_copy(zeros.at[pl.ds(r0, rpt)], acc_sh.at[pl.ds(r0, rpt)])
        pltpu.sync_copy(tbl.at[pl.ds(r0, rpt)], tbl_sh.at[pl.ds(r0, rpt)])
        plsc.subcore_barrier()

        one_v = jnp.ones((16,), jnp.float32)

        def step(j, carry):
            pltpu.async_copy(tbl_sh.at[src_v.at[j]], rows_v, sem).wait()
            pltpu.sync_copy(rows_v, acc_sh.at[dst_v.at[j]], add=True)
            if compute_deg:
                for q in range(_BLK // 16):
                    d = dst_v[j, pl.ds(q * 16, 16)]
                    plsc.addupdate_scatter(
                        hist_v,
                        [lax.shift_right_logical(d, 4),
                         lax.bitwise_and(d, 15)],
                        one_v)
            return carry

        lax.fori_loop(0, k_blocks, step, 0)
        plsc.subcore_barrier()

        pltpu.sync_copy(acc_sh.at[pl.ds(r0, rpt)],
                        out_acc.at[cid, pl.ds(r0, rpt)])
        if compute_deg:
            pltpu.sync_copy(hist_v, out_deg.at[wid])

    return pl.kernel(body, out_type=out_type, mesh=mesh,
                     scratch_types=scratch_types,
                     compiler_params=pltpu.CompilerParams(
                         use_tc_tiling_on_sc=False,
                         needs_layout_passes=False))


def _proj_body(x_ref, wl_ref, wr_ref, p_ref, q_ref):
    xb = x_ref[...]
    p_ref[...] = jnp.dot(xb, wl_ref[...], preferred_element_type=jnp.float32)
    q_ref[...] = jnp.dot(xb, wr_ref[...], preferred_element_type=jnp.float32)


def _h_body(acc_ref, dega_ref, q_ref, b_ref, h_ref, deg_ref):
    a = acc_ref[0] + acc_ref[1]
    dg = jnp.maximum(jnp.sum(dega_ref[...], axis=1), 1.0)[:, None]
    h_ref[...] = jnp.maximum(a / dg + q_ref[...] + b_ref[...], 0.0)
    deg_ref[...] = jnp.broadcast_to(dg, h_ref.shape)


def _out_body(acc_ref, deg_ref, h_ref, wl_ref, wr_ref, b_ref, o_ref):
    mean2 = (acc_ref[0] + acc_ref[1]) / deg_ref[...]
    z = (jnp.dot(mean2, wl_ref[...], preferred_element_type=jnp.float32)
         + jnp.dot(h_ref[...], wr_ref[...], preferred_element_type=jnp.float32)
         + b_ref[...])
    m = jnp.max(z, axis=1, keepdims=True)
    s = jnp.sum(jnp.exp(z - m), axis=1, keepdims=True)
    o_ref[...] = z - m - jnp.log(s)


def kernel(x, edge_index, W1_l, W1_r, b1, W2_l, W2_r, b2):
    n, d_feat = x.shape
    d_hid = W1_l.shape[1]
    n_cls = W2_l.shape[1]
    e = edge_index.shape[1]
    assert d_hid == _D

    src = edge_index[0].astype(jnp.int32)
    dst = edge_index[1].astype(jnp.int32)

    k_blocks = -(-e // (_NW * _BLK))
    e_pad = _NW * k_blocks * _BLK - e
    src_p = jnp.concatenate(
        [src, jnp.zeros((e_pad,), jnp.int32)]).reshape(_NW, k_blocks, _BLK)
    dst_p = jnp.concatenate(
        [dst, jnp.full((e_pad,), n, jnp.int32)]).reshape(_NW, k_blocks, _BLK)

    n_nodes_p = -(-(n + 1) // (_NS * 8)) * (_NS * 8)
    zeros = jnp.zeros((n_nodes_p, _D), jnp.float32)

    sc_scatter_deg = _make_sc_scatter(n_nodes_p, k_blocks, True)
    sc_scatter = _make_sc_scatter(n_nodes_p, k_blocks, False)

    rb = 1000
    grid = (n // rb,)

    p1, q1 = pl.pallas_call(
        _proj_body,
        grid=grid,
        in_specs=[
            pl.BlockSpec((rb, d_feat), lambda i: (i, 0)),
            pl.BlockSpec((d_feat, d_hid), lambda i: (0, 0)),
            pl.BlockSpec((d_feat, d_hid), lambda i: (0, 0)),
        ],
        out_specs=[pl.BlockSpec((rb, d_hid), lambda i: (i, 0))] * 2,
        out_shape=[jax.ShapeDtypeStruct((n, d_hid), jnp.float32)] * 2,
    )(x, W1_l, W1_r)

    tpad = jnp.zeros((n_nodes_p - n, d_hid), jnp.float32)
    acc1, degh = sc_scatter_deg(
        jnp.concatenate([p1, tpad]), src_p, dst_p, zeros)
    acc1 = acc1[:, :n, :]
    dega = degh.reshape(_NW, n_nodes_p)[:, :n].T

    h, deg = pl.pallas_call(
        _h_body,
        grid=grid,
        in_specs=[
            pl.BlockSpec((_NC, rb, d_hid), lambda i: (0, i, 0)),
            pl.BlockSpec((rb, _NW), lambda i: (i, 0)),
            pl.BlockSpec((rb, d_hid), lambda i: (i, 0)),
            pl.BlockSpec((1, d_hid), lambda i: (0, 0)),
        ],
        out_specs=[pl.BlockSpec((rb, d_hid), lambda i: (i, 0))] * 2,
        out_shape=[jax.ShapeDtypeStruct((n, d_hid), jnp.float32)] * 2,
    )(acc1, dega, q1, b1.reshape(1, d_hid))

    acc2 = sc_scatter(jnp.concatenate([h, tpad]), src_p, dst_p, zeros)
    acc2 = acc2[:, :n, :]

    out = pl.pallas_call(
        _out_body,
        grid=grid,
        in_specs=[
            pl.BlockSpec((_NC, rb, d_hid), lambda i: (0, i, 0)),
            pl.BlockSpec((rb, d_hid), lambda i: (i, 0)),
            pl.BlockSpec((rb, d_hid), lambda i: (i, 0)),
            pl.BlockSpec((d_hid, n_cls), lambda i: (0, 0)),
            pl.BlockSpec((d_hid, n_cls), lambda i: (0, 0)),
            pl.BlockSpec((1, n_cls), lambda i: (0, 0)),
        ],
        out_specs=pl.BlockSpec((rb, n_cls), lambda i: (i, 0)),
        out_shape=jax.ShapeDtypeStruct((n, n_cls), jnp.float32),
    )(acc2, deg, h, W2_l, W2_r, b2.reshape(1, n_cls))

    return out


# speedup vs baseline: 4.8253x; 1.3854x over previous
import functools

import jax
import jax.numpy as jnp
from jax import lax
from jax.experimental import pallas as pl
from jax.experimental.pallas import tpu as pltpu
from jax.experimental.pallas import tpu_sc as plsc

_NC = 2
_NS = 16
_NW = _NC * _NS
_BLK = 128
_D = 16


def _make_sc_scatter(n_nodes_p, k_blocks, compute_deg):
    rpt = n_nodes_p // _NS
    hrows = n_nodes_p // 16
    mesh = plsc.VectorSubcoreMesh(
        core_axis_name="c", subcore_axis_name="s",
        num_cores=_NC, num_subcores=_NS)

    acc_t = jax.ShapeDtypeStruct((_NC, n_nodes_p, _D), jnp.float32)
    deg_t = jax.ShapeDtypeStruct((_NW, hrows, 16), jnp.float32)
    out_type = [acc_t, deg_t] if compute_deg else acc_t

    scratch_types = [
        pltpu.VMEM((k_blocks, _BLK), jnp.int32),
        pltpu.VMEM((k_blocks, _BLK), jnp.int32),
        pltpu.VMEM((2, _BLK, _D), jnp.float32),
        pltpu.VMEM((hrows, 16), jnp.float32),
        pltpu.VMEM_SHARED((n_nodes_p, _D), jnp.float32),
        pltpu.VMEM_SHARED((n_nodes_p, _D), jnp.float32),
        pltpu.SemaphoreType.DMA((2,)),
    ]

    def body(tbl, srcb, dstb, zeros, *refs):
        if compute_deg:
            out_acc, out_deg = refs[0], refs[1]
            rest = refs[2:]
        else:
            out_acc, out_deg = refs[0], None
            rest = refs[1:]
        src_v, dst_v, rows_v, hist_v, acc_sh, tbl_sh, sem = rest

        cid = lax.axis_index("c")
        sid = lax.axis_index("s")
        wid = cid * _NS + sid

        pltpu.sync_copy(srcb.at[wid], src_v)
        pltpu.sync_copy(dstb.at[wid], dst_v)
        if compute_deg:
            pltpu.sync_copy(zeros.at[pl.ds(0, hrows)], hist_v)

        r0 = sid * rpt
        pltpu.sync_copy(zeros.at[pl.ds(r0, rpt)], acc_sh.at[pl.ds(r0, rpt)])
        pltpu.sync_copy(tbl.at[pl.ds(r0, rpt)], tbl_sh.at[pl.ds(r0, rpt)])
        plsc.subcore_barrier()

        one_v = jnp.ones((16,), jnp.float32)

        @functools.partial(plsc.parallel_loop, 0, k_blocks, unroll=2)
        def _(j):
            slot = lax.bitwise_and(j, 1)
            buf = rows_v.at[slot]
            pltpu.async_copy(tbl_sh.at[src_v.at[j]], buf, sem.at[slot]).wait()
            pltpu.sync_copy(buf, acc_sh.at[dst_v.at[j]], add=True)
            if compute_deg:
                for q in range(_BLK // 16):
                    d = dst_v[j, pl.ds(q * 16, 16)]
                    plsc.addupdate_scatter(
                        hist_v,
                        [lax.shift_right_logical(d, 4),
                         lax.bitwise_and(d, 15)],
                        one_v)
        plsc.subcore_barrier()

        pltpu.sync_copy(acc_sh.at[pl.ds(r0, rpt)],
                        out_acc.at[cid, pl.ds(r0, rpt)])
        if compute_deg:
            pltpu.sync_copy(hist_v, out_deg.at[wid])

    return pl.kernel(body, out_type=out_type, mesh=mesh,
                     scratch_types=scratch_types,
                     compiler_params=pltpu.CompilerParams(
                         use_tc_tiling_on_sc=False,
                         needs_layout_passes=False))


def _proj_body(x_ref, wl_ref, wr_ref, p_ref, q_ref):
    xb = x_ref[...]
    p_ref[...] = jnp.dot(xb, wl_ref[...], preferred_element_type=jnp.float32)
    q_ref[...] = jnp.dot(xb, wr_ref[...], preferred_element_type=jnp.float32)


def _h_body(acc_ref, dega_ref, q_ref, b_ref, h_ref, deg_ref):
    a = acc_ref[0] + acc_ref[1]
    dg = jnp.maximum(jnp.sum(dega_ref[...], axis=1), 1.0)[:, None]
    h_ref[...] = jnp.maximum(a / dg + q_ref[...] + b_ref[...], 0.0)
    deg_ref[...] = jnp.broadcast_to(dg, h_ref.shape)


def _out_body(acc_ref, deg_ref, h_ref, wl_ref, wr_ref, b_ref, o_ref):
    mean2 = (acc_ref[0] + acc_ref[1]) / deg_ref[...]
    z = (jnp.dot(mean2, wl_ref[...], preferred_element_type=jnp.float32)
         + jnp.dot(h_ref[...], wr_ref[...], preferred_element_type=jnp.float32)
         + b_ref[...])
    m = jnp.max(z, axis=1, keepdims=True)
    s = jnp.sum(jnp.exp(z - m), axis=1, keepdims=True)
    o_ref[...] = z - m - jnp.log(s)


def kernel(x, edge_index, W1_l, W1_r, b1, W2_l, W2_r, b2):
    n, d_feat = x.shape
    d_hid = W1_l.shape[1]
    n_cls = W2_l.shape[1]
    e = edge_index.shape[1]
    assert d_hid == _D

    src = edge_index[0].astype(jnp.int32)
    dst = edge_index[1].astype(jnp.int32)

    k_blocks = -(-e // (_NW * _BLK))
    e_pad = _NW * k_blocks * _BLK - e
    src_p = jnp.concatenate(
        [src, jnp.zeros((e_pad,), jnp.int32)]).reshape(_NW, k_blocks, _BLK)
    dst_p = jnp.concatenate(
        [dst, jnp.full((e_pad,), n, jnp.int32)]).reshape(_NW, k_blocks, _BLK)

    n_nodes_p = -(-(n + 1) // (_NS * 8)) * (_NS * 8)
    zeros = jnp.zeros((n_nodes_p, _D), jnp.float32)

    sc_scatter_deg = _make_sc_scatter(n_nodes_p, k_blocks, True)
    sc_scatter = _make_sc_scatter(n_nodes_p, k_blocks, False)

    rb = 1000
    grid = (n // rb,)

    p1, q1 = pl.pallas_call(
        _proj_body,
        grid=grid,
        in_specs=[
            pl.BlockSpec((rb, d_feat), lambda i: (i, 0)),
            pl.BlockSpec((d_feat, d_hid), lambda i: (0, 0)),
            pl.BlockSpec((d_feat, d_hid), lambda i: (0, 0)),
        ],
        out_specs=[pl.BlockSpec((rb, d_hid), lambda i: (i, 0))] * 2,
        out_shape=[jax.ShapeDtypeStruct((n, d_hid), jnp.float32)] * 2,
    )(x, W1_l, W1_r)

    tpad = jnp.zeros((n_nodes_p - n, d_hid), jnp.float32)
    acc1, degh = sc_scatter_deg(
        jnp.concatenate([p1, tpad]), src_p, dst_p, zeros)
    acc1 = acc1[:, :n, :]
    dega = degh.reshape(_NW, n_nodes_p)[:, :n].T

    h, deg = pl.pallas_call(
        _h_body,
        grid=grid,
        in_specs=[
            pl.BlockSpec((_NC, rb, d_hid), lambda i: (0, i, 0)),
            pl.BlockSpec((rb, _NW), lambda i: (i, 0)),
            pl.BlockSpec((rb, d_hid), lambda i: (i, 0)),
            pl.BlockSpec((1, d_hid), lambda i: (0, 0)),
        ],
        out_specs=[pl.BlockSpec((rb, d_hid), lambda i: (i, 0))] * 2,
        out_shape=[jax.ShapeDtypeStruct((n, d_hid), jnp.float32)] * 2,
    )(acc1, dega, q1, b1.reshape(1, d_hid))

    acc2 = sc_scatter(jnp.concatenate([h, tpad]), src_p, dst_p, zeros)
    acc2 = acc2[:, :n, :]

    out = pl.pallas_call(
        _out_body,
        grid=grid,
        in_specs=[
            pl.BlockSpec((_NC, rb, d_hid), lambda i: (0, i, 0)),
            pl.BlockSpec((rb, d_hid), lambda i: (i, 0)),
            pl.BlockSpec((rb, d_hid), lambda i: (i, 0)),
            pl.BlockSpec((d_hid, n_cls), lambda i: (0, 0)),
            pl.BlockSpec((d_hid, n_cls), lambda i: (0, 0)),
            pl.BlockSpec((1, n_cls), lambda i: (0, 0)),
        ],
        out_specs=pl.BlockSpec((rb, n_cls), lambda i: (i, 0)),
        out_shape=jax.ShapeDtypeStruct((n, n_cls), jnp.float32),
    )(acc2, deg, h, W2_l, W2_r, b2.reshape(1, n_cls))

    return out
